# baseline (speedup 1.0000x reference)
import jax
import jax.numpy as jnp
from jax.experimental import pallas as pl
from jax.experimental.pallas import tpu as pltpu

_EPS = 1e-12


def _netvlad_kernel(x_ref, w_ref, b_ref, c_ref, out_ref):
    xf = x_ref[0]
    w = w_ref[...]
    b = b_ref[...]
    cent = c_ref[...]

    logits = jax.lax.dot_general(
        w, xf, (((1,), (0,)), ((), ())),
        preferred_element_type=jnp.float32) + b
    m = jnp.max(logits, axis=0, keepdims=True)
    e = jnp.exp(logits - m)
    a = e / jnp.sum(e, axis=0, keepdims=True)

    vlad = jax.lax.dot_general(
        a, xf, (((1,), (1,)), ((), ())),
        preferred_element_type=jnp.float32)
    vlad = vlad - jnp.sum(a, axis=1, keepdims=True) * cent

    inorm = jnp.sqrt(jnp.sum(vlad * vlad, axis=1, keepdims=True))
    vlad = vlad / jnp.maximum(inorm, _EPS)
    gnorm = jnp.sqrt(jnp.sum(vlad * vlad))
    out_ref[0] = vlad / jnp.maximum(gnorm, _EPS)


def kernel(x, conv_w, conv_b, centroids):
    N, C, H, W = x.shape
    K = centroids.shape[0]
    P = H * W
    xf = x.reshape(N, C, P)
    b2 = conv_b.reshape(K, 1)

    out = pl.pallas_call(
        _netvlad_kernel,
        grid=(N,),
        in_specs=[
            pl.BlockSpec((1, C, P), lambda n: (n, 0, 0)),
            pl.BlockSpec((K, C), lambda n: (0, 0)),
            pl.BlockSpec((K, 1), lambda n: (0, 0)),
            pl.BlockSpec((K, C), lambda n: (0, 0)),
        ],
        out_specs=pl.BlockSpec((1, K, C), lambda n: (n, 0, 0)),
        out_shape=jax.ShapeDtypeStruct((N, K, C), jnp.float32),
        compiler_params=pltpu.CompilerParams(
            dimension_semantics=("parallel",)),
    )(xf, conv_w, b2, centroids)
    return out.reshape(N, K * C)
